# Initial kernel scaffold; baseline (speedup 1.0000x reference)
import jax
import jax.numpy as jnp
from jax.experimental import pallas as pl


def kernel(indices, table):
    raise NotImplementedError("write your pallas kernel here")


# speedup vs baseline: 1.2846x; 1.2846x over previous
import functools

import jax
import jax.numpy as jnp
from jax import lax
from jax.experimental import pallas as pl
from jax.experimental.pallas import tpu as pltpu
from jax.experimental.pallas import tpu_sc as plsc

VOCAB = 1048576
EMBED_DIM = 128
BATCH = 4096
SEQ = 200

NC = 2
NS = 16
NW = NC * NS

B = BATCH * SEQ
B_PER_W = B // NW
CHUNK = 128
N_CHUNK = B_PER_W // CHUNK


def _gather_body(idx_hbm, table_hbm, out_hbm, idx_v, rows_v, sem):
    cid = lax.axis_index("c")
    sid = lax.axis_index("s")
    wid = sid * NC + cid
    base = wid * B_PER_W

    pltpu.sync_copy(idx_hbm.at[wid], idx_v)

    def chunk(c, carry):
        pltpu.async_copy(table_hbm.at[idx_v.at[c]], rows_v, sem).wait()
        pltpu.sync_copy(rows_v, out_hbm.at[pl.ds(base + c * CHUNK, CHUNK)])
        return carry

    lax.fori_loop(0, N_CHUNK, chunk, 0)


@jax.jit
def _gather(idx3, table):
    kfn = functools.partial(
        pl.kernel,
        out_type=jax.ShapeDtypeStruct((B, EMBED_DIM), jnp.float32),
        mesh=plsc.VectorSubcoreMesh(core_axis_name="c", subcore_axis_name="s"),
        scratch_types=[
            pltpu.VMEM((N_CHUNK, CHUNK), jnp.int32),
            pltpu.VMEM((CHUNK, EMBED_DIM), jnp.float32),
            pltpu.SemaphoreType.DMA,
        ],
    )(_gather_body)
    return kfn(idx3, table)


def kernel(indices, table):
    idx3 = indices.reshape(NW, N_CHUNK, CHUNK).astype(jnp.int32)
    out = _gather(idx3, table)
    return out.reshape(BATCH, SEQ, EMBED_DIM)


# speedup vs baseline: 1.8643x; 1.4513x over previous
import functools

import jax
import jax.numpy as jnp
from jax import lax
from jax.experimental import pallas as pl
from jax.experimental.pallas import tpu as pltpu
from jax.experimental.pallas import tpu_sc as plsc

VOCAB = 1048576
EMBED_DIM = 128
BATCH = 4096
SEQ = 200

NC = 2
NS = 16
NW = NC * NS

B = BATCH * SEQ
B_PER_W = B // NW
CHUNK = 128
N_CHUNK = B_PER_W // CHUNK
U = 2
HALF = U * CHUNK
N_HALF = B_PER_W // HALF
NG = N_HALF // 2


def _gather_body(idx_hbm, table_hbm, out_hbm,
                 idx_v, rows_a, rows_b, gsem_a, gsem_b, osem_a, osem_b):
    cid = lax.axis_index("c")
    sid = lax.axis_index("s")
    wid = sid * NC + cid
    base = wid * B_PER_W

    pltpu.sync_copy(idx_hbm.at[wid], idx_v)

    def gather_desc(h, buf, sem, j):
        return pltpu.make_async_copy(
            table_hbm.at[idx_v.at[h * U + j]],
            buf.at[pl.ds(j * CHUNK, CHUNK)], sem)

    def start_gather(h, buf, sem):
        for j in range(U):
            gather_desc(h, buf, sem, j).start()

    def wait_gather(h, buf, sem):
        for j in range(U):
            gather_desc(h, buf, sem, j).wait()

    def out_desc(h, buf, sem):
        return pltpu.make_async_copy(
            buf, out_hbm.at[pl.ds(base + h * HALF, HALF)], sem)

    def pair(g, lookahead):
        h_a = 2 * g
        h_b = 2 * g + 1
        wait_gather(h_a, rows_a, gsem_a)
        out_desc(h_a, rows_a, osem_a).start()
        start_gather(h_b, rows_b, gsem_b)
        wait_gather(h_b, rows_b, gsem_b)
        out_desc(h_b, rows_b, osem_b).start()
        out_desc(h_a, rows_a, osem_a).wait()
        if lookahead:
            start_gather(h_a + 2, rows_a, gsem_a)
        out_desc(h_b, rows_b, osem_b).wait()

    start_gather(0, rows_a, gsem_a)

    def body(g, carry):
        pair(g, lookahead=True)
        return carry

    lax.fori_loop(0, NG - 1, body, 0)
    pair(NG - 1, lookahead=False)


@jax.jit
def _gather(idx3, table):
    kfn = functools.partial(
        pl.kernel,
        out_type=jax.ShapeDtypeStruct((B, EMBED_DIM), jnp.float32),
        mesh=plsc.VectorSubcoreMesh(core_axis_name="c", subcore_axis_name="s"),
        scratch_types=[
            pltpu.VMEM((N_CHUNK, CHUNK), jnp.int32),
            pltpu.VMEM((HALF, EMBED_DIM), jnp.float32),
            pltpu.VMEM((HALF, EMBED_DIM), jnp.float32),
            pltpu.SemaphoreType.DMA,
            pltpu.SemaphoreType.DMA,
            pltpu.SemaphoreType.DMA,
            pltpu.SemaphoreType.DMA,
        ],
    )(_gather_body)
    return kfn(idx3, table)


def kernel(indices, table):
    idx3 = indices.reshape(NW, N_CHUNK, CHUNK).astype(jnp.int32)
    out = _gather(idx3, table)
    return out.reshape(BATCH, SEQ, EMBED_DIM)


# speedup vs baseline: 1.8733x; 1.0048x over previous
import functools

import jax
import jax.numpy as jnp
from jax import lax
from jax.experimental import pallas as pl
from jax.experimental.pallas import tpu as pltpu
from jax.experimental.pallas import tpu_sc as plsc

VOCAB = 1048576
EMBED_DIM = 128
BATCH = 4096
SEQ = 200

NC = 2
NS = 16
NW = NC * NS

B = BATCH * SEQ
B_PER_W = B // NW
CHUNK = 128
N_CHUNK = B_PER_W // CHUNK
R = 5
NGROUP = N_CHUNK // R


def _gather_body(idx_hbm, table_hbm, out_hbm, idx_v, rows_v,
                 gsem0, gsem1, gsem2, gsem3, gsem4,
                 osem0, osem1, osem2, osem3, osem4):
    gsems = [gsem0, gsem1, gsem2, gsem3, gsem4]
    osems = [osem0, osem1, osem2, osem3, osem4]

    cid = lax.axis_index("c")
    sid = lax.axis_index("s")
    wid = sid * NC + cid
    base = wid * B_PER_W

    pltpu.sync_copy(idx_hbm.at[wid], idx_v)

    def gather_desc(h, p):
        return pltpu.make_async_copy(
            table_hbm.at[idx_v.at[h]], rows_v.at[p], gsems[p])

    def out_desc(h, p):
        return pltpu.make_async_copy(
            rows_v.at[p], out_hbm.at[pl.ds(base + h * CHUNK, CHUNK)], osems[p])

    def pos(h, p, first=False, skip_gather=False):
        gather_desc(h, p).wait()
        out_desc(h, p).start()
        if not first:
            out_desc(h - 1, (p - 1) % R).wait()
        if not skip_gather:
            gather_desc(h + R - 1, (p - 1) % R).start()

    for p in range(R - 1):
        gather_desc(p, p).start()

    for p in range(R):
        pos(p, p, first=(p == 0))

    def body(g, carry):
        for p in range(R):
            pos(g * R + p, p)
        return carry

    lax.fori_loop(1, NGROUP - 1, body, 0)

    for p in range(R):
        h = (NGROUP - 1) * R + p
        pos(h, p, skip_gather=(h + R - 1 >= N_CHUNK))

    out_desc(N_CHUNK - 1, (N_CHUNK - 1) % R).wait()


@jax.jit
def _gather(idx3, table):
    kfn = functools.partial(
        pl.kernel,
        out_type=jax.ShapeDtypeStruct((B, EMBED_DIM), jnp.float32),
        mesh=plsc.VectorSubcoreMesh(core_axis_name="c", subcore_axis_name="s"),
        scratch_types=[
            pltpu.VMEM((N_CHUNK, CHUNK), jnp.int32),
            pltpu.VMEM((R, CHUNK, EMBED_DIM), jnp.float32),
        ] + [pltpu.SemaphoreType.DMA] * (2 * R),
    )(_gather_body)
    return kfn(idx3, table)


def kernel(indices, table):
    idx3 = indices.reshape(NW, N_CHUNK, CHUNK).astype(jnp.int32)
    out = _gather(idx3, table)
    return out.reshape(BATCH, SEQ, EMBED_DIM)
